# baseline (speedup 1.0000x reference)
import jax
import jax.numpy as jnp
from jax import lax
from jax.experimental import pallas as pl
from jax.experimental.pallas import tpu as pltpu
from jax.experimental.pallas import tpu_sc as plsc

_MAX_MZ = 1500.0
_NUM_BINS = 1500
_B = 1024
_T = 200

_NB_PAD = 1504
_NW = 32
_ROWS = _B // _NW
_LANES = 16
_TG = 13
_T_TAIL = _T - (_TG - 1) * _LANES

_PRED_VR = _ROWS * _NUM_BINS // 128
_TGT_VR = _ROWS * _T // 128


_CHUNKS = 8
_CROWS = _ROWS // _CHUNKS


def _sc_body(pred_hbm, mz_hbm, it_hbm,
             dot_hbm, tn2_hbm,
             pred_v, mz_v, it_v, hist_v, idx_v, w_v, dot_v, tn2_v,
             tsem, *psems):
  c = lax.axis_index("c")
  s = lax.axis_index("s")
  wid = s * 2 + c

  base = wid * _ROWS
  cm = pltpu.async_copy(mz_hbm.at[pl.ds(base, _ROWS)], mz_v, tsem)
  ci = pltpu.async_copy(it_hbm.at[pl.ds(base, _ROWS)], it_v, tsem)
  pred_copies = [
      pltpu.async_copy(pred_hbm.at[pl.ds(base + k * _CROWS, _CROWS)],
                       pred_v.at[pl.ds(k * _CROWS, _CROWS)], psems[k])
      for k in range(2)
  ]

  zeros = jnp.zeros((_LANES,), jnp.float32)
  lane = lax.iota(jnp.int32, _LANES)
  tail_mask = lane >= (_LANES - _T_TAIL)

  def zero_hist(j, carry):
    hist_v[pl.ds(j * _LANES, _LANES)] = zeros
    return carry

  lax.fori_loop(0, _NB_PAD // _LANES, zero_hist, 0)
  cm.wait()
  ci.wait()

  def row_body(r, carry):
    rsplat = jnp.full((_LANES,), r, jnp.int32)

    dot = jnp.zeros((_LANES,), jnp.float32)
    for j in range(_TG):
      co = j * _LANES if j < _TG - 1 else _T - _LANES
      mz = mz_v[r, pl.ds(co, _LANES)]
      w = it_v[r, pl.ds(co, _LANES)]
      if j == _TG - 1:
        w = jnp.where(tail_mask, w, 0.0)
      idx = jnp.clip((mz * _MAX_MZ).astype(jnp.int32), 0, _NUM_BINS - 1)
      idx_v[pl.ds(j * _LANES, _LANES)] = idx
      w_v[pl.ds(j * _LANES, _LANES)] = w
      g = plsc.load_gather(pred_v, [rsplat, idx])
      plsc.addupdate_scatter(hist_v, [idx], w)
      dot = dot + g * w

    tn2 = jnp.zeros((_LANES,), jnp.float32)
    for j in range(_TG):
      idx = idx_v[pl.ds(j * _LANES, _LANES)]
      w = w_v[pl.ds(j * _LANES, _LANES)]
      hv = plsc.load_gather(hist_v, [idx])
      tn2 = tn2 + hv * w

    for j in range(_TG):
      idx = idx_v[pl.ds(j * _LANES, _LANES)]
      plsc.store_scatter(hist_v, [idx], zeros)

    dot_v[r // 8, pl.ds((r % 8) * _LANES, _LANES)] = dot
    tn2_v[r // 8, pl.ds((r % 8) * _LANES, _LANES)] = tn2
    return carry

  for k in range(_CHUNKS):
    if k + 2 < _CHUNKS:
      pred_copies.append(
          pltpu.async_copy(
              pred_hbm.at[pl.ds(base + (k + 2) * _CROWS, _CROWS)],
              pred_v.at[pl.ds((k + 2) * _CROWS, _CROWS)], psems[k + 2]))
    pred_copies[k].wait()
    lax.fori_loop(k * _CROWS, (k + 1) * _CROWS, row_body, 0)

  vr = _ROWS * _LANES // 128
  pltpu.sync_copy(dot_v, dot_hbm.at[pl.ds(wid * vr, vr)])
  pltpu.sync_copy(tn2_v, tn2_hbm.at[pl.ds(wid * vr, vr)])


def _pn2_body(pred_ref, out_ref):
  x = pred_ref[...]
  out_ref[...] = jnp.sum(x * x, axis=1, keepdims=True).reshape(16, 8)


def _combine_body(dot_ref, tn2_ref, pn2_ref, out_ref):
  sel = (lax.broadcasted_iota(jnp.int32, (128, 8), 0) // _LANES
         == lax.broadcasted_iota(jnp.int32, (128, 8), 1)
         ).astype(jnp.float32)
  dot = jnp.dot(dot_ref[...], sel, preferred_element_type=jnp.float32)
  tn2 = jnp.dot(tn2_ref[...], sel, preferred_element_type=jnp.float32)
  tnorm = jnp.sqrt(tn2)
  pnorm = jnp.sqrt(pn2_ref[...])
  num = dot / ((pnorm + 1e-8) * (tnorm + 1e-8))
  na = jnp.maximum(pnorm / (pnorm + 1e-8), 1e-8)
  nb = jnp.maximum(tnorm / (tnorm + 1e-8), 1e-8)
  cos = num / (na * nb)
  out_ref[...] = jnp.broadcast_to(1.0 - jnp.mean(cos), (1, 1))


def kernel(pred_binned, target_mz, target_intensity, target_mask):
  sc = pl.kernel(
      _sc_body,
      mesh=plsc.VectorSubcoreMesh(core_axis_name="c", subcore_axis_name="s"),
      compiler_params=pltpu.CompilerParams(
          use_tc_tiling_on_sc=True, needs_layout_passes=False,
          skip_device_barrier=True),
      out_type=(
          jax.ShapeDtypeStruct((_B * _LANES // 128, 128), jnp.float32),
          jax.ShapeDtypeStruct((_B * _LANES // 128, 128), jnp.float32),
      ),
      scratch_types=[
          pltpu.VMEM((_ROWS, _NUM_BINS), jnp.float32),
          pltpu.VMEM((_ROWS, _T), jnp.float32),
          pltpu.VMEM((_ROWS, _T), jnp.float32),
          pltpu.VMEM((_NB_PAD,), jnp.float32),
          pltpu.VMEM((_TG * _LANES,), jnp.int32),
          pltpu.VMEM((_TG * _LANES,), jnp.float32),
          pltpu.VMEM((_ROWS * _LANES // 128, 128), jnp.float32),
          pltpu.VMEM((_ROWS * _LANES // 128, 128), jnp.float32),
      ] + [pltpu.SemaphoreType.DMA] * (_CHUNKS + 1),
  )
  dot, tn2 = sc(pred_binned, target_mz, target_intensity * target_mask)

  pn2 = pl.pallas_call(
      _pn2_body,
      grid=(8,),
      in_specs=[pl.BlockSpec((_B // 8, _NUM_BINS), lambda i: (i, 0))],
      out_specs=pl.BlockSpec((_B // 8 // 8, 8), lambda i: (i, 0)),
      out_shape=jax.ShapeDtypeStruct((_B // 8, 8), jnp.float32),
  )(pred_binned)

  out = pl.pallas_call(
      _combine_body,
      out_shape=jax.ShapeDtypeStruct((1, 1), jnp.float32),
  )(dot, tn2, pn2)
  return out.reshape(())


# speedup vs baseline: 1.0276x; 1.0276x over previous
import jax
import jax.numpy as jnp
from jax import lax
from jax.experimental import pallas as pl
from jax.experimental.pallas import tpu as pltpu
from jax.experimental.pallas import tpu_sc as plsc

_MAX_MZ = 1500.0
_NUM_BINS = 1500
_B = 1024
_T = 200

_NB_PAD = 1504
_NW = 32
_ROWS = _B // _NW
_LANES = 16
_TG = 13
_T_TAIL = _T - (_TG - 1) * _LANES

_PRED_VR = _ROWS * _NUM_BINS // 128
_TGT_VR = _ROWS * _T // 128


_CHUNKS = 4
_CROWS = _ROWS // _CHUNKS


def _sc_body(pred_hbm, mz_hbm, it_hbm,
             dot_hbm, tn2_hbm,
             pred_v, mz_v, it_v, hist_v, idx_v, w_v, dot_v, tn2_v,
             tsem, *psems):
  c = lax.axis_index("c")
  s = lax.axis_index("s")
  wid = s * 2 + c

  base = wid * _ROWS
  cm = pltpu.async_copy(mz_hbm.at[pl.ds(base, _ROWS)], mz_v, tsem)
  ci = pltpu.async_copy(it_hbm.at[pl.ds(base, _ROWS)], it_v, tsem)
  pred_copies = [
      pltpu.async_copy(pred_hbm.at[pl.ds(base + k * _CROWS, _CROWS)],
                       pred_v.at[pl.ds(k * _CROWS, _CROWS)], psems[k])
      for k in range(2)
  ]

  zeros = jnp.zeros((_LANES,), jnp.float32)
  lane = lax.iota(jnp.int32, _LANES)
  tail_mask = lane >= (_LANES - _T_TAIL)

  def zero_hist(j, carry):
    hist_v[pl.ds(j * _LANES, _LANES)] = zeros
    return carry

  lax.fori_loop(0, _NB_PAD // _LANES, zero_hist, 0)
  cm.wait()
  ci.wait()

  def row_body(r, carry):
    rsplat = jnp.full((_LANES,), r, jnp.int32)

    dot = jnp.zeros((_LANES,), jnp.float32)
    for j in range(_TG):
      co = j * _LANES if j < _TG - 1 else _T - _LANES
      mz = mz_v[r, pl.ds(co, _LANES)]
      w = it_v[r, pl.ds(co, _LANES)]
      if j == _TG - 1:
        w = jnp.where(tail_mask, w, 0.0)
      idx = jnp.clip((mz * _MAX_MZ).astype(jnp.int32), 0, _NUM_BINS - 1)
      idx_v[pl.ds(j * _LANES, _LANES)] = idx
      w_v[pl.ds(j * _LANES, _LANES)] = w
      g = plsc.load_gather(pred_v, [rsplat, idx])
      plsc.addupdate_scatter(hist_v, [idx], w)
      dot = dot + g * w

    tn2 = jnp.zeros((_LANES,), jnp.float32)
    for j in range(_TG):
      idx = idx_v[pl.ds(j * _LANES, _LANES)]
      w = w_v[pl.ds(j * _LANES, _LANES)]
      hv = plsc.load_gather(hist_v, [idx])
      tn2 = tn2 + hv * w

    for j in range(_TG):
      idx = idx_v[pl.ds(j * _LANES, _LANES)]
      plsc.store_scatter(hist_v, [idx], zeros)

    dot_v[r // 8, pl.ds((r % 8) * _LANES, _LANES)] = dot
    tn2_v[r // 8, pl.ds((r % 8) * _LANES, _LANES)] = tn2
    return carry

  for k in range(_CHUNKS):
    if k + 2 < _CHUNKS:
      pred_copies.append(
          pltpu.async_copy(
              pred_hbm.at[pl.ds(base + (k + 2) * _CROWS, _CROWS)],
              pred_v.at[pl.ds((k + 2) * _CROWS, _CROWS)], psems[k + 2]))
    pred_copies[k].wait()
    lax.fori_loop(k * _CROWS, (k + 1) * _CROWS, row_body, 0)

  vr = _ROWS * _LANES // 128
  pltpu.sync_copy(dot_v, dot_hbm.at[pl.ds(wid * vr, vr)])
  pltpu.sync_copy(tn2_v, tn2_hbm.at[pl.ds(wid * vr, vr)])


def _pn2_body(pred_ref, out_ref):
  x = pred_ref[...]
  out_ref[...] = jnp.sum(x * x, axis=1, keepdims=True).reshape(16, 8)


def _combine_body(dot_ref, tn2_ref, pn2_ref, out_ref):
  sel = (lax.broadcasted_iota(jnp.int32, (128, 8), 0) // _LANES
         == lax.broadcasted_iota(jnp.int32, (128, 8), 1)
         ).astype(jnp.float32)
  dot = jnp.dot(dot_ref[...], sel, preferred_element_type=jnp.float32)
  tn2 = jnp.dot(tn2_ref[...], sel, preferred_element_type=jnp.float32)
  tnorm = jnp.sqrt(tn2)
  pnorm = jnp.sqrt(pn2_ref[...])
  num = dot / ((pnorm + 1e-8) * (tnorm + 1e-8))
  na = jnp.maximum(pnorm / (pnorm + 1e-8), 1e-8)
  nb = jnp.maximum(tnorm / (tnorm + 1e-8), 1e-8)
  cos = num / (na * nb)
  out_ref[...] = jnp.broadcast_to(1.0 - jnp.mean(cos), (1, 1))


def kernel(pred_binned, target_mz, target_intensity, target_mask):
  sc = pl.kernel(
      _sc_body,
      mesh=plsc.VectorSubcoreMesh(core_axis_name="c", subcore_axis_name="s"),
      compiler_params=pltpu.CompilerParams(
          use_tc_tiling_on_sc=True, needs_layout_passes=False,
          skip_device_barrier=True),
      out_type=(
          jax.ShapeDtypeStruct((_B * _LANES // 128, 128), jnp.float32),
          jax.ShapeDtypeStruct((_B * _LANES // 128, 128), jnp.float32),
      ),
      scratch_types=[
          pltpu.VMEM((_ROWS, _NUM_BINS), jnp.float32),
          pltpu.VMEM((_ROWS, _T), jnp.float32),
          pltpu.VMEM((_ROWS, _T), jnp.float32),
          pltpu.VMEM((_NB_PAD,), jnp.float32),
          pltpu.VMEM((_TG * _LANES,), jnp.int32),
          pltpu.VMEM((_TG * _LANES,), jnp.float32),
          pltpu.VMEM((_ROWS * _LANES // 128, 128), jnp.float32),
          pltpu.VMEM((_ROWS * _LANES // 128, 128), jnp.float32),
      ] + [pltpu.SemaphoreType.DMA] * (_CHUNKS + 1),
  )
  dot, tn2 = sc(pred_binned, target_mz, target_intensity * target_mask)

  pn2 = pl.pallas_call(
      _pn2_body,
      grid=(8,),
      in_specs=[pl.BlockSpec((_B // 8, _NUM_BINS), lambda i: (i, 0))],
      out_specs=pl.BlockSpec((_B // 8 // 8, 8), lambda i: (i, 0)),
      out_shape=jax.ShapeDtypeStruct((_B // 8, 8), jnp.float32),
  )(pred_binned)

  out = pl.pallas_call(
      _combine_body,
      out_shape=jax.ShapeDtypeStruct((1, 1), jnp.float32),
  )(dot, tn2, pn2)
  return out.reshape(())
